# baseline (speedup 1.0000x reference)
import functools

import jax
import jax.numpy as jnp
from jax import lax
from jax.experimental import pallas as pl
from jax.experimental.pallas import tpu as pltpu
from jax.experimental.pallas import tpu_sc as plsc

NC = 2
NS = 16
NW = NC * NS

V = 1000000
D = 32
VCHUNK = 512
N_FULL = V // VCHUNK
ROWS_PER_CHUNK = VCHUNK * D // 128
VR = V * D // 128 + 16
GRP = 64
NBUF = 8


def _rsqrt(x):
    i = plsc.bitcast(x, jnp.int32)
    i = 0x5F3759DF - lax.shift_right_logical(i, 1)
    y = plsc.bitcast(i, jnp.float32)
    xh = x * 0.5
    for _ in range(3):
        y = y * (1.5 - xh * y * y)
    return y


def _make_transpose():
    mesh = plsc.VectorSubcoreMesh(
        core_axis_name="c", subcore_axis_name="s", num_cores=NC, num_subcores=NS
    )
    n_iter = (N_FULL + NW - 1) // NW

    @functools.partial(
        pl.kernel,
        out_type=jax.ShapeDtypeStruct((VR, 128), jnp.float32),
        mesh=mesh,
        scratch_types=[
            pltpu.VMEM((D, VCHUNK), jnp.float32),
            pltpu.VMEM((D, VCHUNK), jnp.float32),
            pltpu.VMEM((ROWS_PER_CHUNK, 128), jnp.float32),
            pltpu.VMEM((ROWS_PER_CHUNK, 128), jnp.float32),
            pltpu.SemaphoreType.DMA,
            pltpu.SemaphoreType.DMA,
            pltpu.SemaphoreType.DMA,
            pltpu.SemaphoreType.DMA,
        ],
        compiler_params=pltpu.CompilerParams(needs_layout_passes=False),
    )
    def transpose_kernel(wt_hbm, out_hbm, inA, inB, trA, trB,
                         isemA, isemB, osemA, osemB):
        wid = lax.axis_index("s") * NC + lax.axis_index("c")
        iota = lax.iota(jnp.int32, 16)

        def fire_in(c, buf, sem):
            v0 = pl.multiple_of(c * VCHUNK, VCHUNK)
            return pltpu.async_copy(
                wt_hbm.at[:, pl.ds(v0, VCHUNK)], buf, sem)

        def wait_in(buf, sem):
            pltpu.make_async_copy(
                wt_hbm.at[:, pl.ds(0, VCHUNK)], buf, sem).wait()

        def fire_out(c, buf, sem):
            r0 = pl.multiple_of(c * ROWS_PER_CHUNK, ROWS_PER_CHUNK)
            return pltpu.async_copy(
                buf, out_hbm.at[pl.ds(r0, ROWS_PER_CHUNK)], sem)

        def wait_out(buf, sem):
            pltpu.make_async_copy(
                buf, out_hbm.at[pl.ds(0, ROWS_PER_CHUNK)], sem).wait()

        def transpose_chunk(in_buf, tr_buf):
            def vv_body(q, carry):
                rows8 = [jnp.full((16,), q * 8 + s, jnp.int32)
                         for s in range(8)]
                for k in range(32):
                    vsplat = jnp.full((16,), q * 32 + k, jnp.int32)
                    for d0 in (0, 16):
                        fs = k * D + d0
                        x = plsc.load_gather(in_buf, [d0 + iota, vsplat])
                        plsc.store_scatter(
                            tr_buf,
                            [rows8[fs // 128], (fs % 128) + iota], x)
                return carry

            lax.fori_loop(0, VCHUNK // 32, vv_body, 0)

        def pair_body(p, carry):
            i = 2 * p
            c = i * NW + wid
            c2 = (i + 1) * NW + wid

            @pl.when(c < N_FULL)
            def _():
                wait_in(inA, isemA)

                @pl.when(p > 0)
                def _():
                    wait_out(trA, osemA)

                transpose_chunk(inA, trA)

                @pl.when(c2 + NW < N_FULL)
                def _():
                    fire_in(c2 + NW, inA, isemA)

                fire_out(c, trA, osemA)

            @pl.when(c2 < N_FULL)
            def _():
                wait_in(inB, isemB)

                @pl.when(p > 0)
                def _():
                    wait_out(trB, osemB)

                transpose_chunk(inB, trB)

                @pl.when(c2 + 2 * NW < N_FULL)
                def _():
                    fire_in(c2 + 2 * NW, inB, isemB)

                fire_out(c2, trB, osemB)
            return carry

        @pl.when(wid < N_FULL)
        def _():
            fire_in(wid, inA, isemA)

        @pl.when(wid + NW < N_FULL)
        def _():
            fire_in(wid + NW, inB, isemB)

        n_pairs = (n_iter + 1) // 2
        lax.fori_loop(0, n_pairs, pair_body, 0)

        @pl.when(wid < N_FULL)
        def _():
            wait_out(trA, osemA)

        @pl.when(wid + NW < N_FULL)
        def _():
            wait_out(trB, osemB)

        @pl.when(wid == 0)
        def _():
            v0 = pl.multiple_of(N_FULL * VCHUNK, 128)
            pltpu.async_copy(
                wt_hbm.at[:, pl.ds(v0, 128)],
                inA.at[:, pl.ds(0, 128)], isemA).wait()

            def tail_body(q, carry):
                rows8 = [jnp.full((16,), q * 8 + s, jnp.int32)
                         for s in range(8)]
                for k in range(32):
                    vsplat = jnp.full((16,), q * 32 + k, jnp.int32)
                    for d0 in (0, 16):
                        fs = k * D + d0
                        x = plsc.load_gather(inA, [d0 + iota, vsplat])
                        plsc.store_scatter(
                            trA, [rows8[fs // 128], (fs % 128) + iota], x)
                return carry

            lax.fori_loop(0, 4, tail_body, 0)
            r0 = pl.multiple_of(N_FULL * ROWS_PER_CHUNK, 32)
            pltpu.async_copy(
                trA.at[pl.ds(0, 32)],
                out_hbm.at[pl.ds(r0, 32)], osemA).wait()

    return transpose_kernel


def _make_gather(B, BATCH, FIELDS):
    b_per_w = BATCH // NW
    n_per_w = b_per_w * FIELDS
    assert b_per_w * D // 16 % 32 == 0

    mesh = plsc.VectorSubcoreMesh(
        core_axis_name="c", subcore_axis_name="s", num_cores=NC, num_subcores=NS
    )

    @functools.partial(
        pl.kernel,
        out_type=jax.ShapeDtypeStruct((FIELDS, D, BATCH), jnp.float32),
        mesh=mesh,
        scratch_types=[
            pltpu.VMEM((n_per_w,), jnp.int32),
            pltpu.VMEM((n_per_w,), jnp.int32),
            pltpu.VMEM((NBUF * GRP, 128), jnp.float32),
            pltpu.VMEM((D, b_per_w), jnp.float32),
            pltpu.SemaphoreType.DMA((NBUF,)),
            pltpu.SemaphoreType.DMA,
        ],
        compiler_params=pltpu.CompilerParams(needs_layout_passes=False),
    )
    def gather_kernel(idx4_hbm, off_hbm, table_hbm, out_hbm,
                      idx4_v, off_v, rows, out_v, gsems, wsem):
        wid = lax.axis_index("s") * NC + lax.axis_index("c")
        base = wid * n_per_w
        b0 = wid * b_per_w
        iota = lax.iota(jnp.int32, 16)
        n_chunks = FIELDS * (b_per_w // GRP)

        pltpu.sync_copy(idx4_hbm.at[pl.ds(base, n_per_w)], idx4_v)
        pltpu.sync_copy(off_hbm.at[pl.ds(base, n_per_w)], off_v)

        def fire_gather(c, slot):
            s = pl.multiple_of(c * GRP, GRP)
            return pltpu.async_copy(
                table_hbm.at[idx4_v.at[pl.ds(s, GRP)]],
                rows.at[pl.ds(slot * GRP, GRP)], gsems.at[slot])

        def wait_gather(slot):
            pltpu.make_async_copy(
                table_hbm.at[pl.ds(0, GRP)],
                rows.at[pl.ds(0, GRP)], gsems.at[slot]).wait()

        for j in range(NBUF):
            fire_gather(j, j)

        cpf = b_per_w // GRP

        def body(c, carry):
            slot = lax.rem(c, NBUF)
            j = lax.rem(c, cpf)
            f = lax.div(c, cpf)

            @pl.when(jnp.logical_and(j == 0, f > 0))
            def _():
                pltpu.make_async_copy(
                    out_v, out_hbm.at[0, :, pl.ds(0, b_per_w)], wsem).wait()

            wait_gather(slot)
            rbase = slot * GRP
            for g in range(GRP // 16):
                off = off_v[pl.ds(c * GRP + g * 16, 16)]
                r = iota + rbase + g * 16
                vs = []
                acc = jnp.zeros((16,), jnp.float32)
                for jj in range(D):
                    v = plsc.load_gather(rows, [r, off + jj])
                    vs.append(v)
                    acc = acc + v * v
                inv = _rsqrt(jnp.maximum(acc, 1e-24))
                col = j * GRP + g * 16
                for jj in range(D):
                    out_v[jj, pl.ds(col, 16)] = vs[jj] * inv

            @pl.when(c + NBUF < n_chunks)
            def _():
                fire_gather(c + NBUF, slot)

            @pl.when(j == cpf - 1)
            def _():
                pltpu.async_copy(
                    out_v, out_hbm.at[f, :, pl.ds(b0, b_per_w)], wsem)
            return carry

        lax.fori_loop(0, n_chunks, body, 0)
        pltpu.make_async_copy(
            out_v, out_hbm.at[0, :, pl.ds(0, b_per_w)], wsem).wait()

    return gather_kernel


def kernel(input, W):
    batch, fields = input.shape
    Vw, Dw = W.shape
    B = batch * fields
    idx = input.reshape(NW, batch // NW, fields).transpose(0, 2, 1).reshape(B)
    idx = idx.astype(jnp.int32)
    idx4 = lax.shift_right_logical(idx, 2)
    off = lax.shift_left(jnp.bitwise_and(idx, 3), 5)
    wt = W.T
    table = _make_transpose()(wt)
    out = _make_gather(B, batch, fields)(idx4, off, table)
    return out.transpose(2, 0, 1)


# speedup vs baseline: 2.5208x; 2.5208x over previous
import functools

import jax
import jax.numpy as jnp
from jax import lax
from jax.experimental import pallas as pl
from jax.experimental.pallas import tpu as pltpu
from jax.experimental.pallas import tpu_sc as plsc

NC = 2
NS = 16
NW = NC * NS

V = 1000000
D = 32
VCHUNK = 512
N_FULL = V // VCHUNK
ROWS_PER_CHUNK = VCHUNK * D // 128
VR = V * D // 128 + 16
GRP = 64
NBUF = 8


def _rsqrt(x):
    i = plsc.bitcast(x, jnp.int32)
    i = 0x5F3759DF - lax.shift_right_logical(i, 1)
    y = plsc.bitcast(i, jnp.float32)
    xh = x * 0.5
    for _ in range(3):
        y = y * (1.5 - xh * y * y)
    return y


def _make_transpose():
    mesh = plsc.VectorSubcoreMesh(
        core_axis_name="c", subcore_axis_name="s", num_cores=NC, num_subcores=NS
    )
    n_iter = (N_FULL + NW - 1) // NW

    @functools.partial(
        pl.kernel,
        out_type=jax.ShapeDtypeStruct((VR, 128), jnp.float32),
        mesh=mesh,
        scratch_types=[
            pltpu.VMEM((D, VCHUNK), jnp.float32),
            pltpu.VMEM((D, VCHUNK), jnp.float32),
            pltpu.VMEM((ROWS_PER_CHUNK, 128), jnp.float32),
            pltpu.VMEM((ROWS_PER_CHUNK, 128), jnp.float32),
            pltpu.SemaphoreType.DMA,
            pltpu.SemaphoreType.DMA,
            pltpu.SemaphoreType.DMA,
            pltpu.SemaphoreType.DMA,
        ],
        compiler_params=pltpu.CompilerParams(needs_layout_passes=False),
    )
    def transpose_kernel(wt_hbm, out_hbm, inA, inB, trA, trB,
                         isemA, isemB, osemA, osemB):
        wid = lax.axis_index("s") * NC + lax.axis_index("c")
        iota = lax.iota(jnp.int32, 16)

        def fire_in(c, buf, sem):
            v0 = pl.multiple_of(c * VCHUNK, VCHUNK)
            return pltpu.async_copy(
                wt_hbm.at[:, pl.ds(v0, VCHUNK)], buf, sem)

        def wait_in(buf, sem):
            pltpu.make_async_copy(
                wt_hbm.at[:, pl.ds(0, VCHUNK)], buf, sem).wait()

        def fire_out(c, buf, sem):
            r0 = pl.multiple_of(c * ROWS_PER_CHUNK, ROWS_PER_CHUNK)
            return pltpu.async_copy(
                buf, out_hbm.at[pl.ds(r0, ROWS_PER_CHUNK)], sem)

        def wait_out(buf, sem):
            pltpu.make_async_copy(
                buf, out_hbm.at[pl.ds(0, ROWS_PER_CHUNK)], sem).wait()

        def transpose_chunk(in_buf, tr_buf):
            def vv_body(q, carry):
                vv0 = q * 16
                for d0 in (0, 16):
                    for jj in range(16):
                        vvec = vv0 + jnp.bitwise_and(jj + iota, 15)
                        x = plsc.load_gather(in_buf, [d0 + iota, vvec])
                        flat = vvec * D + (d0 + iota)
                        plsc.store_scatter(
                            tr_buf,
                            [lax.shift_right_logical(flat, 7),
                             jnp.bitwise_and(flat, 127)], x)
                return carry

            lax.fori_loop(0, VCHUNK // 16, vv_body, 0)

        def pair_body(p, carry):
            i = 2 * p
            c = i * NW + wid
            c2 = (i + 1) * NW + wid

            @pl.when(c < N_FULL)
            def _():
                wait_in(inA, isemA)

                @pl.when(p > 0)
                def _():
                    wait_out(trA, osemA)

                transpose_chunk(inA, trA)

                @pl.when(c2 + NW < N_FULL)
                def _():
                    fire_in(c2 + NW, inA, isemA)

                fire_out(c, trA, osemA)

            @pl.when(c2 < N_FULL)
            def _():
                wait_in(inB, isemB)

                @pl.when(p > 0)
                def _():
                    wait_out(trB, osemB)

                transpose_chunk(inB, trB)

                @pl.when(c2 + 2 * NW < N_FULL)
                def _():
                    fire_in(c2 + 2 * NW, inB, isemB)

                fire_out(c2, trB, osemB)
            return carry

        @pl.when(wid < N_FULL)
        def _():
            fire_in(wid, inA, isemA)

        @pl.when(wid + NW < N_FULL)
        def _():
            fire_in(wid + NW, inB, isemB)

        n_pairs = (n_iter + 1) // 2
        lax.fori_loop(0, n_pairs, pair_body, 0)

        @pl.when(wid < N_FULL)
        def _():
            wait_out(trA, osemA)

        @pl.when(wid + NW < N_FULL)
        def _():
            wait_out(trB, osemB)

        @pl.when(wid == 0)
        def _():
            v0 = pl.multiple_of(N_FULL * VCHUNK, 128)
            pltpu.async_copy(
                wt_hbm.at[:, pl.ds(v0, 128)],
                inA.at[:, pl.ds(0, 128)], isemA).wait()

            def tail_body(q, carry):
                vv0 = q * 16
                for d0 in (0, 16):
                    for jj in range(16):
                        vvec = vv0 + jnp.bitwise_and(jj + iota, 15)
                        x = plsc.load_gather(inA, [d0 + iota, vvec])
                        flat = vvec * D + (d0 + iota)
                        plsc.store_scatter(
                            trA,
                            [lax.shift_right_logical(flat, 7),
                             jnp.bitwise_and(flat, 127)], x)
                return carry

            lax.fori_loop(0, 8, tail_body, 0)
            r0 = pl.multiple_of(N_FULL * ROWS_PER_CHUNK, 32)
            pltpu.async_copy(
                trA.at[pl.ds(0, 32)],
                out_hbm.at[pl.ds(r0, 32)], osemA).wait()

    return transpose_kernel


def _make_gather(B, BATCH, FIELDS):
    b_per_w = BATCH // NW
    n_per_w = b_per_w * FIELDS
    assert b_per_w * D // 16 % 32 == 0

    mesh = plsc.VectorSubcoreMesh(
        core_axis_name="c", subcore_axis_name="s", num_cores=NC, num_subcores=NS
    )

    @functools.partial(
        pl.kernel,
        out_type=jax.ShapeDtypeStruct((FIELDS, D, BATCH), jnp.float32),
        mesh=mesh,
        scratch_types=[
            pltpu.VMEM((n_per_w,), jnp.int32),
            pltpu.VMEM((n_per_w,), jnp.int32),
            pltpu.VMEM((NBUF * GRP, 128), jnp.float32),
            pltpu.VMEM((D, b_per_w), jnp.float32),
            pltpu.SemaphoreType.DMA((NBUF,)),
            pltpu.SemaphoreType.DMA,
        ],
        compiler_params=pltpu.CompilerParams(needs_layout_passes=False),
    )
    def gather_kernel(idx4_hbm, off_hbm, table_hbm, out_hbm,
                      idx4_v, off_v, rows, out_v, gsems, wsem):
        wid = lax.axis_index("s") * NC + lax.axis_index("c")
        base = wid * n_per_w
        b0 = wid * b_per_w
        iota = lax.iota(jnp.int32, 16)
        n_chunks = FIELDS * (b_per_w // GRP)

        pltpu.sync_copy(idx4_hbm.at[pl.ds(base, n_per_w)], idx4_v)
        pltpu.sync_copy(off_hbm.at[pl.ds(base, n_per_w)], off_v)

        def fire_gather(c, slot):
            s = pl.multiple_of(c * GRP, GRP)
            return pltpu.async_copy(
                table_hbm.at[idx4_v.at[pl.ds(s, GRP)]],
                rows.at[pl.ds(slot * GRP, GRP)], gsems.at[slot])

        def wait_gather(slot):
            pltpu.make_async_copy(
                table_hbm.at[pl.ds(0, GRP)],
                rows.at[pl.ds(0, GRP)], gsems.at[slot]).wait()

        for j in range(NBUF):
            fire_gather(j, j)

        cpf = b_per_w // GRP

        def body(c, carry):
            slot = lax.rem(c, NBUF)
            j = lax.rem(c, cpf)
            f = lax.div(c, cpf)

            @pl.when(jnp.logical_and(j == 0, f > 0))
            def _():
                pltpu.make_async_copy(
                    out_v, out_hbm.at[0, :, pl.ds(0, b_per_w)], wsem).wait()

            wait_gather(slot)
            rbase = slot * GRP
            for g in range(GRP // 16):
                off = off_v[pl.ds(c * GRP + g * 16, 16)]
                r = iota + rbase + g * 16
                vs = []
                acc = jnp.zeros((16,), jnp.float32)
                for jj in range(D):
                    dvec = jnp.bitwise_and(jj + iota, D - 1)
                    v = plsc.load_gather(rows, [r, off + dvec])
                    vs.append(v)
                    acc = acc + v * v
                inv = _rsqrt(jnp.maximum(acc, 1e-24))
                col = j * GRP + g * 16
                for jj in range(D):
                    dvec = jnp.bitwise_and(jj + iota, D - 1)
                    plsc.store_scatter(
                        out_v, [dvec, col + iota], vs[jj] * inv)

            @pl.when(c + NBUF < n_chunks)
            def _():
                fire_gather(c + NBUF, slot)

            @pl.when(j == cpf - 1)
            def _():
                pltpu.async_copy(
                    out_v, out_hbm.at[f, :, pl.ds(b0, b_per_w)], wsem)
            return carry

        lax.fori_loop(0, n_chunks, body, 0)
        pltpu.make_async_copy(
            out_v, out_hbm.at[0, :, pl.ds(0, b_per_w)], wsem).wait()

    return gather_kernel


def kernel(input, W):
    batch, fields = input.shape
    Vw, Dw = W.shape
    B = batch * fields
    idx = input.reshape(NW, batch // NW, fields).transpose(0, 2, 1).reshape(B)
    idx = idx.astype(jnp.int32)
    idx4 = lax.shift_right_logical(idx, 2)
    off = lax.shift_left(jnp.bitwise_and(idx, 3), 5)
    wt = W.T
    table = _make_transpose()(wt)
    out = _make_gather(B, batch, fields)(idx4, off, table)
    return out.transpose(2, 0, 1)
